# baseline (speedup 1.0000x reference)
import functools

import jax
import jax.numpy as jnp
from jax import lax
from jax.experimental import pallas as pl
from jax.experimental.pallas import tpu as pltpu
from jax.experimental.pallas import tpu_sc as plsc

B = 4096
L = 50
D = 32
LANES = 16
NW = 32
BPW = B // NW
NROWS = 100000
TILES = 16
RPT = NROWS // TILES
G = 4
NGRP = BPW // G

MASKHI = -65536


def _embed_kernel_body(u_idx, i_idx, h_idx, u_tab, i_tab, h_pk,
                       out_u, out_i, out_h,
                       uidx_v, iidx_v, hraw, urows, irows, acc,
                       gbuf0, gbuf1, spmem_tab,
                       sem_idx, sem_ui, sem_h, sem_tab):
    cid = lax.axis_index("c")
    sid = lax.axis_index("s")
    wid = sid * 2 + cid
    base = wid * BPW

    trow = sid * RPT
    cp_t = pltpu.async_copy(h_pk.at[pl.ds(trow, RPT)],
                            spmem_tab.at[pl.ds(trow, RPT)], sem_tab)

    cp_u = pltpu.async_copy(u_idx.at[pl.ds(base, BPW)], uidx_v, sem_idx)
    cp_i = pltpu.async_copy(i_idx.at[pl.ds(base, BPW)], iidx_v, sem_idx)
    cp_h = pltpu.async_copy(h_idx.at[wid], hraw, sem_idx)

    cp_u.wait()
    cp_i.wait()

    g_u = pltpu.async_copy(u_tab.at[uidx_v], urows, sem_ui)
    g_i = pltpu.async_copy(i_tab.at[iidx_v], irows, sem_ui)

    cp_h.wait()
    cp_t.wait()
    plsc.subcore_barrier()

    scale = jnp.full((LANES,), 1.0 / L, jnp.float32)

    def fire_group(g0v, gb):
        return [
            pltpu.async_copy(spmem_tab.at[hraw.at[g0v + b]],
                             gb.at[pl.ds(b * L, L)], sem_h)
            for b in range(G)
        ]

    def accum_group(g0v, gb):
        def body(b, carry):
            acc_a = jnp.zeros((LANES,), jnp.float32)
            acc_b = jnp.zeros((LANES,), jnp.float32)
            for j in range(L):
                w = gb[b * L + j, pl.ds(0, LANES)]
                acc_a = acc_a + lax.bitcast_convert_type(w & MASKHI,
                                                         jnp.float32)
                acc_b = acc_b + lax.bitcast_convert_type(
                    lax.shift_left(w, 16), jnp.float32)
            acc[g0v + b, pl.ds(0, LANES)] = acc_a * scale
            acc[g0v + b, pl.ds(LANES, LANES)] = acc_b * scale
            return carry

        lax.fori_loop(0, G, body, 0)

    def pair(it, carry):
        ga = it * 2 * G
        gb_ = ga + G
        cps0 = fire_group(ga, gbuf0)
        cps1 = fire_group(gb_, gbuf1)
        for cp in cps0:
            cp.wait()
        accum_group(ga, gbuf0)
        for cp in cps1:
            cp.wait()
        accum_group(gb_, gbuf1)
        return carry

    lax.fori_loop(0, NGRP // 2, pair, 0)

    g_u.wait()
    g_i.wait()
    st_u = pltpu.async_copy(urows, out_u.at[pl.ds(base, BPW)], sem_ui)
    st_i = pltpu.async_copy(irows, out_i.at[pl.ds(base, BPW)], sem_ui)
    pltpu.sync_copy(acc, out_h.at[pl.ds(base, BPW)])
    st_u.wait()
    st_i.wait()


@jax.jit
def kernel(user_idx, item_idx, hist_idx, user_table, item_table, hist_table):
    u_idx = user_idx.reshape(B).astype(jnp.int32)
    i_idx = item_idx.reshape(B).astype(jnp.int32)
    h_idx = hist_idx.astype(jnp.int32).reshape(NW, BPW, L)

    tb = lax.bitcast_convert_type(hist_table.astype(jnp.bfloat16)
                                  .astype(jnp.float32), jnp.uint32)
    h_pk = lax.bitcast_convert_type(
        (tb[:, :LANES] & jnp.uint32(0xFFFF0000)) | (tb[:, LANES:] >> 16),
        jnp.int32)

    mesh = plsc.VectorSubcoreMesh(core_axis_name="c", subcore_axis_name="s")
    run = functools.partial(
        pl.kernel,
        out_type=[jax.ShapeDtypeStruct((B, D), jnp.float32),
                  jax.ShapeDtypeStruct((B, D), jnp.float32),
                  jax.ShapeDtypeStruct((B, D), jnp.float32)],
        mesh=mesh,
        compiler_params=pltpu.CompilerParams(use_tc_tiling_on_sc=False),
        scratch_types=[
            pltpu.VMEM((BPW,), jnp.int32),
            pltpu.VMEM((BPW,), jnp.int32),
            pltpu.VMEM((BPW, L), jnp.int32),
            pltpu.VMEM((BPW, D), jnp.float32),
            pltpu.VMEM((BPW, D), jnp.float32),
            pltpu.VMEM((BPW, D), jnp.float32),
            pltpu.VMEM((G * L, LANES), jnp.int32),
            pltpu.VMEM((G * L, LANES), jnp.int32),
            pltpu.VMEM_SHARED((NROWS, LANES), jnp.int32),
            pltpu.SemaphoreType.DMA,
            pltpu.SemaphoreType.DMA,
            pltpu.SemaphoreType.DMA,
            pltpu.SemaphoreType.DMA,
        ],
    )(_embed_kernel_body)

    e_u, e_i, e_h = run(u_idx, i_idx, h_idx, user_table, item_table, h_pk)
    return jnp.stack([e_u, e_i, e_h], axis=1)


# speedup vs baseline: 1.0906x; 1.0906x over previous
import functools

import jax
import jax.numpy as jnp
from jax import lax
from jax.experimental import pallas as pl
from jax.experimental.pallas import tpu as pltpu
from jax.experimental.pallas import tpu_sc as plsc

B = 4096
L = 50
D = 32
LANES = 16
NW = 32
BPW = B // NW
GCH = 7


def _embed_kernel_body(u_idx, i_idx, h_idx, u_tab, i_tab, h_tab,
                       out_u, out_i, out_h,
                       uidx_v, iidx_v, hidx_v, urows, irows, acc,
                       sem_ui, sem_h):
    cid = lax.axis_index("c")
    sid = lax.axis_index("s")
    wid = sid * 2 + cid
    base = wid * BPW

    pltpu.sync_copy(u_idx.at[pl.ds(base, BPW)], uidx_v)
    pltpu.sync_copy(i_idx.at[pl.ds(base, BPW)], iidx_v)
    pltpu.sync_copy(h_idx.at[pl.ds(wid * L, L)], hidx_v)

    g_u = pltpu.async_copy(u_tab.at[uidx_v], urows, sem_ui)
    g_i = pltpu.async_copy(i_tab.at[iidx_v], irows, sem_ui)

    pltpu.async_copy(h_tab.at[hidx_v.at[0]], acc, sem_h).wait()

    def add_group(g, carry):
        cps = [
            pltpu.async_copy(h_tab.at[hidx_v.at[1 + g * GCH + k]],
                             acc, sem_h, add=True)
            for k in range(GCH)
        ]
        for cp in cps:
            cp.wait()
        return carry

    lax.fori_loop(0, (L - 1) // GCH, add_group, 0)

    scale = jnp.full((LANES,), 1.0 / L, jnp.float32)

    def scl(r, carry):
        acc[r, pl.ds(0, LANES)] = acc[r, pl.ds(0, LANES)] * scale
        acc[r, pl.ds(LANES, LANES)] = acc[r, pl.ds(LANES, LANES)] * scale
        return carry

    lax.fori_loop(0, BPW, scl, 0)

    g_u.wait()
    g_i.wait()
    pltpu.sync_copy(urows, out_u.at[pl.ds(base, BPW)])
    pltpu.sync_copy(irows, out_i.at[pl.ds(base, BPW)])
    pltpu.sync_copy(acc, out_h.at[pl.ds(base, BPW)])


@jax.jit
def kernel(user_idx, item_idx, hist_idx, user_table, item_table, hist_table):
    u_idx = user_idx.reshape(B).astype(jnp.int32)
    i_idx = item_idx.reshape(B).astype(jnp.int32)
    h_idx = (hist_idx.astype(jnp.int32)
             .reshape(NW, BPW, L)
             .transpose(0, 2, 1)
             .reshape(NW * L, BPW))

    mesh = plsc.VectorSubcoreMesh(core_axis_name="c", subcore_axis_name="s")
    run = functools.partial(
        pl.kernel,
        out_type=[jax.ShapeDtypeStruct((B, D), jnp.float32),
                  jax.ShapeDtypeStruct((B, D), jnp.float32),
                  jax.ShapeDtypeStruct((B, D), jnp.float32)],
        mesh=mesh,
        compiler_params=pltpu.CompilerParams(use_tc_tiling_on_sc=False),
        scratch_types=[
            pltpu.VMEM((BPW,), jnp.int32),
            pltpu.VMEM((BPW,), jnp.int32),
            pltpu.VMEM((L, BPW), jnp.int32),
            pltpu.VMEM((BPW, D), jnp.float32),
            pltpu.VMEM((BPW, D), jnp.float32),
            pltpu.VMEM((BPW, D), jnp.float32),
            pltpu.SemaphoreType.DMA,
            pltpu.SemaphoreType.DMA,
        ],
    )(_embed_kernel_body)

    e_u, e_i, e_h = run(u_idx, i_idx, h_idx, user_table, item_table,
                        hist_table)
    return jnp.stack([e_u, e_i, e_h], axis=1)


# speedup vs baseline: 1.0966x; 1.0055x over previous
import functools

import jax
import jax.numpy as jnp
from jax import lax
from jax.experimental import pallas as pl
from jax.experimental.pallas import tpu as pltpu
from jax.experimental.pallas import tpu_sc as plsc

B = 4096
L = 50
D = 32
LANES = 16
NW = 32
BPW = B // NW
GCH = 12
NG = (L - 1) // GCH


def _embed_kernel_body(u_idx, i_idx, h_idx, u_tab, i_tab, h_tab,
                       out_u, out_i, out_h,
                       uidx_v, iidx_v, hidx_v, urows, irows, acc,
                       sem_ui, sem_h):
    cid = lax.axis_index("c")
    sid = lax.axis_index("s")
    wid = sid * 2 + cid
    base = wid * BPW

    pltpu.sync_copy(u_idx.at[pl.ds(base, BPW)], uidx_v)
    pltpu.sync_copy(i_idx.at[pl.ds(base, BPW)], iidx_v)
    pltpu.sync_copy(h_idx.at[pl.ds(wid * L, L)], hidx_v)

    g_u = pltpu.async_copy(u_tab.at[uidx_v], urows, sem_ui)
    g_i = pltpu.async_copy(i_tab.at[iidx_v], irows, sem_ui)

    pltpu.async_copy(h_tab.at[hidx_v.at[0]], acc, sem_h).wait()
    pltpu.async_copy(h_tab.at[hidx_v.at[L - 1]], acc, sem_h, add=True)

    def fire_group(g):
        for k in range(GCH):
            pltpu.async_copy(h_tab.at[hidx_v.at[1 + g * GCH + k]],
                             acc, sem_h, add=True)

    def drain(n):
        for _ in range(n):
            pltpu.make_async_copy(h_tab.at[pl.ds(0, BPW)], acc,
                                  sem_h).wait()

    fire_group(0)

    def pipelined(g, carry):
        fire_group(g)
        drain(GCH)
        return carry

    lax.fori_loop(1, NG, pipelined, 0)
    drain(GCH + 1)

    scale = jnp.full((LANES,), 1.0 / L, jnp.float32)

    def scl(r, carry):
        acc[r, pl.ds(0, LANES)] = acc[r, pl.ds(0, LANES)] * scale
        acc[r, pl.ds(LANES, LANES)] = acc[r, pl.ds(LANES, LANES)] * scale
        return carry

    lax.fori_loop(0, BPW, scl, 0)

    g_u.wait()
    g_i.wait()
    pltpu.sync_copy(urows, out_u.at[pl.ds(base, BPW)])
    pltpu.sync_copy(irows, out_i.at[pl.ds(base, BPW)])
    pltpu.sync_copy(acc, out_h.at[pl.ds(base, BPW)])


@jax.jit
def kernel(user_idx, item_idx, hist_idx, user_table, item_table, hist_table):
    u_idx = user_idx.reshape(B).astype(jnp.int32)
    i_idx = item_idx.reshape(B).astype(jnp.int32)
    h_idx = (hist_idx.astype(jnp.int32)
             .reshape(NW, BPW, L)
             .transpose(0, 2, 1)
             .reshape(NW * L, BPW))

    mesh = plsc.VectorSubcoreMesh(core_axis_name="c", subcore_axis_name="s")
    run = functools.partial(
        pl.kernel,
        out_type=[jax.ShapeDtypeStruct((B, D), jnp.float32),
                  jax.ShapeDtypeStruct((B, D), jnp.float32),
                  jax.ShapeDtypeStruct((B, D), jnp.float32)],
        mesh=mesh,
        compiler_params=pltpu.CompilerParams(use_tc_tiling_on_sc=False),
        scratch_types=[
            pltpu.VMEM((BPW,), jnp.int32),
            pltpu.VMEM((BPW,), jnp.int32),
            pltpu.VMEM((L, BPW), jnp.int32),
            pltpu.VMEM((BPW, D), jnp.float32),
            pltpu.VMEM((BPW, D), jnp.float32),
            pltpu.VMEM((BPW, D), jnp.float32),
            pltpu.SemaphoreType.DMA,
            pltpu.SemaphoreType.DMA,
        ],
    )(_embed_kernel_body)

    e_u, e_i, e_h = run(u_idx, i_idx, h_idx, user_table, item_table,
                        hist_table)
    return jnp.stack([e_u, e_i, e_h], axis=1)
